# baseline (speedup 1.0000x reference)
import functools

import jax
import jax.numpy as jnp
from jax import lax
from jax.experimental import pallas as pl
from jax.experimental.pallas import tpu as pltpu
from jax.experimental.pallas import tpu_sc as plsc

N = 100_000
E = 1_600_000
H = 32
F0 = 35

TW = 48
MW = 32
WW = 8

NODE_BLK = 1024
N_BLOCKS = 98
NP = NODE_BLK * N_BLOCKS
PAD_IDX = N

NC, NS = 2, 16
NWRK = NC * NS

GB = 256
WBLKS = 196
EPW = WBLKS * GB
E_PAD = NWRK * EPW
EB = E_PAD // GB
GCH = 2
GITERS = WBLKS // GCH

TBLKS = EB // NS

HALF = N // 2
STRIPE = HALF // NS

EBLK = 2048
EGRID = E_PAD // EBLK

_mesh = plsc.VectorSubcoreMesh(core_axis_name="c", subcore_axis_name="s",
                               num_cores=NC, num_subcores=NS)

_DIM0 = (((0,), (0,)), ((), ()))


def _full(shape):
    nd = len(shape)
    return pl.BlockSpec(shape, lambda i, _n=nd: (0,) * _n)


def _front_body(nf_t, xyz_t, w1, b1, w2, b2, wemb, bemb, t_out):
    z = lax.dot_general(nf_t[:, :], w1[:, :], _DIM0) + b1[:, :]
    h = jnp.where(z > 0, z, jnp.exp(jnp.minimum(z, 0.0)) - 1.0)
    h = h @ w2[:, :] + b2[:, :]
    h = h @ wemb[:, :] + bemb[:, :]
    t_out[:, 0:H] = h
    t_out[:, H:H + 3] = xyz_t[:, :].T
    t_out[:, H + 3:TW] = jnp.zeros((NODE_BLK, TW - H - 3), jnp.float32)


def _front_call(nf_t, xyz_t, w1, b1, w2, b2, wemb, bemb):
    return pl.pallas_call(
        _front_body,
        grid=(N_BLOCKS,),
        in_specs=[
            pl.BlockSpec((F0, NODE_BLK), lambda i: (0, i)),
            pl.BlockSpec((3, NODE_BLK), lambda i: (0, i)),
            _full(w1.shape), _full(b1.shape), _full(w2.shape),
            _full(b2.shape), _full(wemb.shape), _full(bemb.shape),
        ],
        out_specs=pl.BlockSpec((NODE_BLK, TW), lambda i: (i, 0)),
        out_shape=jax.ShapeDtypeStruct((NP, TW), jnp.float32),
    )(nf_t, xyz_t, w1, b1, w2, b2, wemb, bemb)


def _edge_body(gs, gd, ea_t, ew1h, ew1d, ew1r, ew1a, eb1, ew2, eb2,
               cw1, cb1, cw2, m_out, w_out):
    hs = gs[:, 0:H]
    hd = gd[:, 0:H]
    diff = gs[:, H:H + 3] - gd[:, H:H + 3]
    radial = jnp.sum(diff * diff, axis=1, keepdims=True)
    t = (hs @ ew1h[:, :] + hd @ ew1d[:, :] + radial @ ew1r[:, :]
         + lax.dot_general(ea_t[:, :], ew1a[:, :], _DIM0) + eb1[:, :])
    m = jax.nn.silu(t)
    m = jax.nn.silu(m @ ew2[:, :] + eb2[:, :])
    c = jax.nn.silu(m @ cw1[:, :] + cb1[:, :])
    cm = c @ cw2[:, :]
    m_out[:, :] = m
    w_out[:, 0:3] = diff * cm
    w_out[:, 3:4] = jnp.ones((EBLK, 1), jnp.float32)
    w_out[:, 4:WW] = jnp.zeros((EBLK, WW - 4), jnp.float32)


def _edge_call(gs, gd, ea_t, ew1h, ew1d, ew1r, ew1a, eb1, ew2, eb2,
               cw1, cb1, cw2):
    return pl.pallas_call(
        _edge_body,
        grid=(EGRID,),
        in_specs=[
            pl.BlockSpec((EBLK, TW), lambda i: (i, 0)),
            pl.BlockSpec((EBLK, TW), lambda i: (i, 0)),
            pl.BlockSpec((2, EBLK), lambda i: (0, i)),
            _full(ew1h.shape), _full(ew1d.shape), _full(ew1r.shape),
            _full(ew1a.shape), _full(eb1.shape), _full(ew2.shape),
            _full(eb2.shape), _full(cw1.shape), _full(cb1.shape),
            _full(cw2.shape),
        ],
        out_specs=[pl.BlockSpec((EBLK, MW), lambda i: (i, 0)),
                   pl.BlockSpec((EBLK, WW), lambda i: (i, 0))],
        out_shape=[jax.ShapeDtypeStruct((E_PAD, MW), jnp.float32),
                   jax.ShapeDtypeStruct((E_PAD, WW), jnp.float32)],
    )(gs, gd, ea_t, ew1h, ew1d, ew1r, ew1a, eb1, ew2, eb2, cw1, cb1, cw2)


def _node_body(t_in, acc_m, acc_w, nw1, nb1, nw2, nb2, t_out):
    h = t_in[:, 0:H]
    x = t_in[:, H:H + 3]
    msum = acc_m[:, :]
    wsum = acc_w[:, 0:3]
    cnt = acc_w[:, 3:4]
    x2 = x + wsum / jnp.maximum(cnt, 1.0)
    nin = jnp.concatenate([h, msum], axis=1)
    o = jax.nn.silu(nin @ nw1[:, :] + nb1[:, :]) @ nw2[:, :] + nb2[:, :]
    t_out[:, 0:H] = h + o
    t_out[:, H:H + 3] = x2
    t_out[:, H + 3:TW] = jnp.zeros((NODE_BLK, TW - H - 3), jnp.float32)


def _node_call(t, acc_m, acc_w, nw1, nb1, nw2, nb2):
    return pl.pallas_call(
        _node_body,
        grid=(N_BLOCKS,),
        in_specs=[
            pl.BlockSpec((NODE_BLK, TW), lambda i: (i, 0)),
            pl.BlockSpec((NODE_BLK, MW), lambda i: (i, 0)),
            pl.BlockSpec((NODE_BLK, WW), lambda i: (i, 0)),
            _full(nw1.shape), _full(nb1.shape), _full(nw2.shape),
            _full(nb2.shape),
        ],
        out_specs=pl.BlockSpec((NODE_BLK, TW), lambda i: (i, 0)),
        out_shape=jax.ShapeDtypeStruct((NP, TW), jnp.float32),
    )(t, acc_m, acc_w, nw1, nb1, nw2, nb2)


def _out_body(t_in, wout, bout, o_ref):
    o_ref[:, :] = t_in[:, 0:H] @ wout[:, :] + bout[:, :]


def _out_call(t, wout, bout):
    return pl.pallas_call(
        _out_body,
        grid=(N_BLOCKS,),
        in_specs=[
            pl.BlockSpec((NODE_BLK, TW), lambda i: (i, 0)),
            _full(wout.shape), _full(bout.shape),
        ],
        out_specs=pl.BlockSpec((NODE_BLK, H), lambda i: (i, 0)),
        out_shape=jax.ShapeDtypeStruct((N, H), jnp.float32),
    )(t, wout, bout)


@functools.partial(
    pl.kernel,
    out_type=(jax.ShapeDtypeStruct((E_PAD, TW), jnp.float32),
              jax.ShapeDtypeStruct((E_PAD, TW), jnp.float32)),
    mesh=_mesh,
    scratch_types=[
        pltpu.VMEM((2, GCH, GB), jnp.int32),
        pltpu.VMEM((2, GCH, GB), jnp.int32),
        pltpu.VMEM((2, GCH * GB, TW), jnp.float32),
        pltpu.VMEM((2, GCH * GB, TW), jnp.float32),
        pltpu.SemaphoreType.DMA,
        pltpu.SemaphoreType.DMA,
        pltpu.SemaphoreType.DMA,
        pltpu.SemaphoreType.DMA,
        pltpu.SemaphoreType.DMA,
    ],
    compiler_params=pltpu.CompilerParams(use_tc_tiling_on_sc=False),
)
def _gather_k(t_hbm, src_hbm, dst_hbm, gs_hbm, gd_hbm,
              sidx, didx, gsb, gdb, sem_i0, sem_i1, sem_g, sem_o0, sem_o1):
    c = lax.axis_index("c")
    s = lax.axis_index("s")
    w = s * NC + c
    blk0 = w * WBLKS
    sem_i = (sem_i0, sem_i1)
    sem_o = (sem_o0, sem_o1)

    def fire_idx(it, b):
        b0 = blk0 + it * GCH
        pltpu.async_copy(src_hbm.at[pl.ds(b0, GCH)], sidx.at[b], sem_i[b])
        pltpu.async_copy(dst_hbm.at[pl.ds(b0, GCH)], didx.at[b], sem_i[b])

    fire_idx(0, 0)

    def body(g, carry):
        for b in (0, 1):
            b2 = 1 - b
            it = 2 * g + b
            pltpu.make_async_copy(
                src_hbm.at[pl.ds(0, GCH)], sidx.at[b], sem_i[b]).wait()
            pltpu.make_async_copy(
                dst_hbm.at[pl.ds(0, GCH)], didx.at[b], sem_i[b]).wait()

            @pl.when(g > 0)
            def _wait_out():
                pltpu.make_async_copy(
                    gsb.at[b], gs_hbm.at[pl.ds(0, GCH * GB)], sem_o[b]).wait()
                pltpu.make_async_copy(
                    gdb.at[b], gd_hbm.at[pl.ds(0, GCH * GB)], sem_o[b]).wait()

            descs = []
            for j in range(GCH):
                descs.append(pltpu.async_copy(
                    t_hbm.at[sidx.at[b].at[j]],
                    gsb.at[b].at[pl.ds(j * GB, GB)], sem_g))
                descs.append(pltpu.async_copy(
                    t_hbm.at[didx.at[b].at[j]],
                    gdb.at[b].at[pl.ds(j * GB, GB)], sem_g))

            if b == 0:
                fire_idx(it + 1, b2)
            else:
                @pl.when(g < GITERS // 2 - 1)
                def _fire_next():
                    fire_idx(it + 1, b2)

            for d in descs:
                d.wait()

            e0 = (blk0 + it * GCH) * GB
            pltpu.async_copy(gsb.at[b], gs_hbm.at[pl.ds(e0, GCH * GB)],
                             sem_o[b])
            pltpu.async_copy(gdb.at[b], gd_hbm.at[pl.ds(e0, GCH * GB)],
                             sem_o[b])
        return carry

    lax.fori_loop(0, GITERS // 2, body, 0)

    for b in (0, 1):
        pltpu.make_async_copy(
            gsb.at[b], gs_hbm.at[pl.ds(0, GCH * GB)], sem_o[b]).wait()
        pltpu.make_async_copy(
            gdb.at[b], gd_hbm.at[pl.ds(0, GCH * GB)], sem_o[b]).wait()


def _make_scatter(width, gch):
    siters = TBLKS // gch
    s2 = siters // 2

    @functools.partial(
        pl.kernel,
        out_type=jax.ShapeDtypeStruct((N, width), jnp.float32),
        mesh=_mesh,
        scratch_types=[
            pltpu.VMEM((2, gch, GB), jnp.int32),
            pltpu.VMEM((2, gch, GB), jnp.int32),
            pltpu.VMEM((2, gch * GB, width), jnp.float32),
            pltpu.VMEM_SHARED((HALF + 8, width), jnp.float32),
            pltpu.SemaphoreType.DMA,
            pltpu.SemaphoreType.DMA,
            pltpu.SemaphoreType.DMA,
            pltpu.SemaphoreType.DMA,
        ],
        compiler_params=pltpu.CompilerParams(use_tc_tiling_on_sc=False),
    )
    def _scatter_k(m_hbm, src_hbm, zeros_hbm, acc_hbm,
                   sidx, lidx, mbuf, accsh, sem_i0, sem_i1, sem_s0, sem_s1):
        c = lax.axis_index("c")
        s = lax.axis_index("s")
        base = c * HALF
        sem_i = (sem_i0, sem_i1)
        sem_s = (sem_s0, sem_s1)

        pltpu.sync_copy(zeros_hbm.at[pl.ds(0, STRIPE)],
                        accsh.at[pl.ds(s * STRIPE, STRIPE)])

        @pl.when(s == NS - 1)
        def _zero_trash():
            pltpu.sync_copy(zeros_hbm.at[pl.ds(0, 8)],
                            accsh.at[pl.ds(HALF, 8)])

        plsc.subcore_barrier()

        def fire_loads(it, b):
            b0 = s * TBLKS + it * gch
            pltpu.async_copy(src_hbm.at[pl.ds(b0, gch)], sidx.at[b], sem_i[b])
            pltpu.async_copy(m_hbm.at[pl.ds(b0 * GB, gch * GB)],
                             mbuf.at[b], sem_i[b])

        def drain_scatter(b):
            for j in range(gch):
                pltpu.make_async_copy(
                    mbuf.at[b].at[pl.ds(j * GB, GB)],
                    accsh.at[pl.ds(0, GB)], sem_s[b]).wait()

        fire_loads(0, 0)

        def body(g, carry):
            for b in (0, 1):
                b2 = 1 - b
                it = 2 * g + b
                pltpu.make_async_copy(
                    src_hbm.at[pl.ds(0, gch)], sidx.at[b], sem_i[b]).wait()
                pltpu.make_async_copy(
                    m_hbm.at[pl.ds(0, gch * GB)], mbuf.at[b], sem_i[b]).wait()

                for j in range(gch):
                    for k in range(GB // 16):
                        v = sidx[b, j, pl.ds(k * 16, 16)]
                        lo = v - base
                        ok = (lo >= 0) & (lo < HALF)
                        lidx[b, j, pl.ds(k * 16, 16)] = jnp.where(ok, lo, HALF)

                if b == 1:
                    drain_scatter(b2)
                else:
                    @pl.when(g > 0)
                    def _drain_prev():
                        drain_scatter(b2)

                for j in range(gch):
                    pltpu.async_copy(
                        mbuf.at[b].at[pl.ds(j * GB, GB)],
                        accsh.at[lidx.at[b].at[j]], sem_s[b], add=True)

                if b == 0:
                    fire_loads(it + 1, b2)
                else:
                    @pl.when(g < s2 - 1)
                    def _fire_next():
                        fire_loads(it + 1, b2)
            return carry

        lax.fori_loop(0, s2, body, 0)

        drain_scatter(1)

        plsc.subcore_barrier()
        pltpu.sync_copy(accsh.at[pl.ds(s * STRIPE, STRIPE)],
                        acc_hbm.at[pl.ds(base + s * STRIPE, STRIPE)])

    return _scatter_k


_scatter_m = _make_scatter(MW, 1)
_scatter_w = _make_scatter(WW, 4)


def kernel(node_feat, coords, edge_attr, edge_index,
           W1, W2, Wemb, Wout, b1, b2, bemb, bout,
           cW1_0, cW1_1, cW2_0, cW2_1, cb1_0, cb1_1,
           eW1_0, eW1_1, eW2_0, eW2_1, eb1_0, eb1_1, eb2_0, eb2_1,
           nW1_0, nW1_1, nW2_0, nW2_1, nb1_0, nb1_1, nb2_0, nb2_1):
    nf_t = jnp.pad(node_feat.T, ((0, 0), (0, NP - N)))
    xyz_t = jnp.pad(coords.T, ((0, 0), (0, NP - N)))
    src_p = jnp.pad(edge_index[0], (0, E_PAD - E),
                    constant_values=PAD_IDX).reshape(EB, GB)
    dst_p = jnp.pad(edge_index[1], (0, E_PAD - E),
                    constant_values=PAD_IDX).reshape(EB, GB)
    ea_t = jnp.pad(edge_attr.T, ((0, 0), (0, E_PAD - E)))
    zeros_m = jnp.zeros((STRIPE, MW), jnp.float32)
    zeros_w = jnp.zeros((STRIPE, WW), jnp.float32)

    def row(b):
        return b.reshape(1, -1)

    t = _front_call(nf_t, xyz_t, W1, row(b1), W2, row(b2), Wemb, row(bemb))

    layers = [
        (eW1_0, eb1_0, eW2_0, eb2_0, cW1_0, cb1_0, cW2_0,
         nW1_0, nb1_0, nW2_0, nb2_0),
        (eW1_1, eb1_1, eW2_1, eb2_1, cW1_1, cb1_1, cW2_1,
         nW1_1, nb1_1, nW2_1, nb2_1),
    ]
    for (ew1, eb1, ew2, eb2, cw1, cb1, cw2, nw1, nb1, nw2, nb2) in layers:
        gs, gd = _gather_k(t, src_p, dst_p)
        m, w = _edge_call(gs, gd, ea_t,
                          ew1[0:H], ew1[H:2 * H], ew1[2 * H:2 * H + 1],
                          ew1[2 * H + 1:], row(eb1), ew2, row(eb2),
                          cw1, row(cb1), cw2)
        acc_m = _scatter_m(m, src_p, zeros_m)
        acc_w = _scatter_w(w, src_p, zeros_w)
        t = _node_call(t, acc_m, acc_w, nw1, row(nb1), nw2, row(nb2))

    return _out_call(t, Wout, row(bout))


# speedup vs baseline: 1.2225x; 1.2225x over previous
import functools

import jax
import jax.numpy as jnp
from jax import lax
from jax.experimental import pallas as pl
from jax.experimental.pallas import tpu as pltpu
from jax.experimental.pallas import tpu_sc as plsc

N = 100_000
E = 1_600_000
H = 32
F0 = 35

TW = 48
MW = 32
WW = 8

NODE_BLK = 1024
N_BLOCKS = 98
NP = NODE_BLK * N_BLOCKS
PAD_IDX = N

NC, NS = 2, 16
NWRK = NC * NS

GB = 256
WBLKS = 196
EPW = WBLKS * GB
E_PAD = NWRK * EPW
EB = E_PAD // GB

WBLKS_H = WBLKS // 2
EH = E_PAD // 2
HB = EB // 2
GITERS = WBLKS_H

TBLKS_H = HB // NS

HALF = N // 2
STRIPE = HALF // NS

EBLK = 4096
EGRID_H = EH // EBLK

_mesh = plsc.VectorSubcoreMesh(core_axis_name="c", subcore_axis_name="s",
                               num_cores=NC, num_subcores=NS)

_DIM0 = (((0,), (0,)), ((), ()))


def _full(shape):
    nd = len(shape)
    return pl.BlockSpec(shape, lambda i, _n=nd: (0,) * _n)


def _front_body(nf_t, xyz_t, w1, b1, w2, b2, wemb, bemb, t_out):
    z = lax.dot_general(nf_t[:, :], w1[:, :], _DIM0) + b1[:, :]
    h = jnp.where(z > 0, z, jnp.exp(jnp.minimum(z, 0.0)) - 1.0)
    h = h @ w2[:, :] + b2[:, :]
    h = h @ wemb[:, :] + bemb[:, :]
    t_out[:, 0:H] = h
    t_out[:, H:H + 3] = xyz_t[:, :].T
    t_out[:, H + 3:TW] = jnp.zeros((NODE_BLK, TW - H - 3), jnp.float32)


def _front_call(nf_t, xyz_t, w1, b1, w2, b2, wemb, bemb):
    return pl.pallas_call(
        _front_body,
        grid=(N_BLOCKS,),
        in_specs=[
            pl.BlockSpec((F0, NODE_BLK), lambda i: (0, i)),
            pl.BlockSpec((3, NODE_BLK), lambda i: (0, i)),
            _full(w1.shape), _full(b1.shape), _full(w2.shape),
            _full(b2.shape), _full(wemb.shape), _full(bemb.shape),
        ],
        out_specs=pl.BlockSpec((NODE_BLK, TW), lambda i: (i, 0)),
        out_shape=jax.ShapeDtypeStruct((NP, TW), jnp.float32),
    )(nf_t, xyz_t, w1, b1, w2, b2, wemb, bemb)


def _edge_body(gs, gd, ea_t, ew1h, ew1d, ew1r, ew1a, eb1, ew2, eb2,
               cw1, cb1, cw2, m_out, w_out):
    hs = gs[:, 0:H]
    hd = gd[:, 0:H]
    diff = gs[:, H:H + 3] - gd[:, H:H + 3]
    radial = jnp.sum(diff * diff, axis=1, keepdims=True)
    t = (hs @ ew1h[:, :] + hd @ ew1d[:, :] + radial @ ew1r[:, :]
         + lax.dot_general(ea_t[:, :], ew1a[:, :], _DIM0) + eb1[:, :])
    m = jax.nn.silu(t)
    m = jax.nn.silu(m @ ew2[:, :] + eb2[:, :])
    c = jax.nn.silu(m @ cw1[:, :] + cb1[:, :])
    cm = c @ cw2[:, :]
    m_out[:, :] = m
    w_out[:, 0:3] = diff * cm
    w_out[:, 3:4] = jnp.ones((EBLK, 1), jnp.float32)
    w_out[:, 4:WW] = jnp.zeros((EBLK, WW - 4), jnp.float32)


def _edge_call(gs, gd, ea_t, ew1h, ew1d, ew1r, ew1a, eb1, ew2, eb2,
               cw1, cb1, cw2):
    return pl.pallas_call(
        _edge_body,
        grid=(EGRID_H,),
        in_specs=[
            pl.BlockSpec((EBLK, TW), lambda i: (i, 0)),
            pl.BlockSpec((EBLK, TW), lambda i: (i, 0)),
            pl.BlockSpec((2, EBLK), lambda i: (0, i)),
            _full(ew1h.shape), _full(ew1d.shape), _full(ew1r.shape),
            _full(ew1a.shape), _full(eb1.shape), _full(ew2.shape),
            _full(eb2.shape), _full(cw1.shape), _full(cb1.shape),
            _full(cw2.shape),
        ],
        out_specs=[pl.BlockSpec((EBLK, MW), lambda i: (i, 0)),
                   pl.BlockSpec((EBLK, WW), lambda i: (i, 0))],
        out_shape=[jax.ShapeDtypeStruct((EH, MW), jnp.float32),
                   jax.ShapeDtypeStruct((EH, WW), jnp.float32)],
    )(gs, gd, ea_t, ew1h, ew1d, ew1r, ew1a, eb1, ew2, eb2, cw1, cb1, cw2)


def _node_body(t_in, acc_ma, acc_mb, acc_wa, acc_wb, nw1, nb1, nw2, nb2,
               t_out):
    h = t_in[:, 0:H]
    x = t_in[:, H:H + 3]
    msum = acc_ma[:, :] + acc_mb[:, :]
    acc_w = acc_wa[:, :] + acc_wb[:, :]
    wsum = acc_w[:, 0:3]
    cnt = acc_w[:, 3:4]
    x2 = x + wsum / jnp.maximum(cnt, 1.0)
    nin = jnp.concatenate([h, msum], axis=1)
    o = jax.nn.silu(nin @ nw1[:, :] + nb1[:, :]) @ nw2[:, :] + nb2[:, :]
    t_out[:, 0:H] = h + o
    t_out[:, H:H + 3] = x2
    t_out[:, H + 3:TW] = jnp.zeros((NODE_BLK, TW - H - 3), jnp.float32)


def _node_call(t, acc_ma, acc_mb, acc_wa, acc_wb, nw1, nb1, nw2, nb2):
    return pl.pallas_call(
        _node_body,
        grid=(N_BLOCKS,),
        in_specs=[
            pl.BlockSpec((NODE_BLK, TW), lambda i: (i, 0)),
            pl.BlockSpec((NODE_BLK, MW), lambda i: (i, 0)),
            pl.BlockSpec((NODE_BLK, MW), lambda i: (i, 0)),
            pl.BlockSpec((NODE_BLK, WW), lambda i: (i, 0)),
            pl.BlockSpec((NODE_BLK, WW), lambda i: (i, 0)),
            _full(nw1.shape), _full(nb1.shape), _full(nw2.shape),
            _full(nb2.shape),
        ],
        out_specs=pl.BlockSpec((NODE_BLK, TW), lambda i: (i, 0)),
        out_shape=jax.ShapeDtypeStruct((NP, TW), jnp.float32),
    )(t, acc_ma, acc_mb, acc_wa, acc_wb, nw1, nb1, nw2, nb2)


def _out_body(t_in, wout, bout, o_ref):
    o_ref[:, :] = t_in[:, 0:H] @ wout[:, :] + bout[:, :]


def _out_call(t, wout, bout):
    return pl.pallas_call(
        _out_body,
        grid=(N_BLOCKS,),
        in_specs=[
            pl.BlockSpec((NODE_BLK, TW), lambda i: (i, 0)),
            _full(wout.shape), _full(bout.shape),
        ],
        out_specs=pl.BlockSpec((NODE_BLK, H), lambda i: (i, 0)),
        out_shape=jax.ShapeDtypeStruct((N, H), jnp.float32),
    )(t, wout, bout)


@functools.partial(
    pl.kernel,
    out_type=(jax.ShapeDtypeStruct((EH, TW), jnp.float32),
              jax.ShapeDtypeStruct((EH, TW), jnp.float32)),
    mesh=_mesh,
    scratch_types=[
        pltpu.VMEM((2, 1, GB), jnp.int32),
        pltpu.VMEM((2, 1, GB), jnp.int32),
        pltpu.VMEM((2, GB, TW), jnp.float32),
        pltpu.VMEM((2, GB, TW), jnp.float32),
        pltpu.SemaphoreType.DMA,
        pltpu.SemaphoreType.DMA,
        pltpu.SemaphoreType.DMA,
        pltpu.SemaphoreType.DMA,
        pltpu.SemaphoreType.DMA,
    ],
    compiler_params=pltpu.CompilerParams(use_tc_tiling_on_sc=False),
)
def _gather_k(t_hbm, src_hbm, dst_hbm, gs_hbm, gd_hbm,
              sidx, didx, gsb, gdb, sem_i0, sem_i1, sem_g, sem_o0, sem_o1):
    c = lax.axis_index("c")
    s = lax.axis_index("s")
    w = s * NC + c
    blk0 = w * WBLKS_H
    sem_i = (sem_i0, sem_i1)
    sem_o = (sem_o0, sem_o1)

    def fire_idx(it, b):
        b0 = blk0 + it
        pltpu.async_copy(src_hbm.at[pl.ds(b0, 1)], sidx.at[b], sem_i[b])
        pltpu.async_copy(dst_hbm.at[pl.ds(b0, 1)], didx.at[b], sem_i[b])

    fire_idx(0, 0)

    def body(g, carry):
        for b in (0, 1):
            b2 = 1 - b
            it = 2 * g + b
            pltpu.make_async_copy(
                src_hbm.at[pl.ds(0, 1)], sidx.at[b], sem_i[b]).wait()
            pltpu.make_async_copy(
                dst_hbm.at[pl.ds(0, 1)], didx.at[b], sem_i[b]).wait()

            @pl.when(g > 0)
            def _wait_out():
                pltpu.make_async_copy(
                    gsb.at[b], gs_hbm.at[pl.ds(0, GB)], sem_o[b]).wait()
                pltpu.make_async_copy(
                    gdb.at[b], gd_hbm.at[pl.ds(0, GB)], sem_o[b]).wait()

            d0 = pltpu.async_copy(
                t_hbm.at[sidx.at[b].at[0]], gsb.at[b], sem_g)
            d1 = pltpu.async_copy(
                t_hbm.at[didx.at[b].at[0]], gdb.at[b], sem_g)

            if b == 0:
                fire_idx(it + 1, b2)
            else:
                @pl.when(g < GITERS // 2 - 1)
                def _fire_next():
                    fire_idx(it + 1, b2)

            d0.wait()
            d1.wait()

            e0 = (blk0 + it) * GB
            pltpu.async_copy(gsb.at[b], gs_hbm.at[pl.ds(e0, GB)], sem_o[b])
            pltpu.async_copy(gdb.at[b], gd_hbm.at[pl.ds(e0, GB)], sem_o[b])
        return carry

    lax.fori_loop(0, GITERS // 2, body, 0)

    for b in (0, 1):
        pltpu.make_async_copy(
            gsb.at[b], gs_hbm.at[pl.ds(0, GB)], sem_o[b]).wait()
        pltpu.make_async_copy(
            gdb.at[b], gd_hbm.at[pl.ds(0, GB)], sem_o[b]).wait()


def _make_scatter(width, gch):
    siters = TBLKS_H // gch
    s2 = siters // 2

    @functools.partial(
        pl.kernel,
        out_type=jax.ShapeDtypeStruct((N, width), jnp.float32),
        mesh=_mesh,
        scratch_types=[
            pltpu.VMEM((2, gch, GB), jnp.int32),
            pltpu.VMEM((2, gch, GB), jnp.int32),
            pltpu.VMEM((2, gch * GB, width), jnp.float32),
            pltpu.VMEM_SHARED((HALF + 8, width), jnp.float32),
            pltpu.SemaphoreType.DMA,
            pltpu.SemaphoreType.DMA,
            pltpu.SemaphoreType.DMA,
            pltpu.SemaphoreType.DMA,
        ],
        compiler_params=pltpu.CompilerParams(use_tc_tiling_on_sc=False),
    )
    def _scatter_k(m_hbm, src_hbm, zeros_hbm, acc_hbm,
                   sidx, lidx, mbuf, accsh, sem_i0, sem_i1, sem_s0, sem_s1):
        c = lax.axis_index("c")
        s = lax.axis_index("s")
        base = c * HALF
        sem_i = (sem_i0, sem_i1)
        sem_s = (sem_s0, sem_s1)

        pltpu.sync_copy(zeros_hbm.at[pl.ds(0, STRIPE)],
                        accsh.at[pl.ds(s * STRIPE, STRIPE)])

        @pl.when(s == NS - 1)
        def _zero_trash():
            pltpu.sync_copy(zeros_hbm.at[pl.ds(0, 8)],
                            accsh.at[pl.ds(HALF, 8)])

        plsc.subcore_barrier()

        def fire_loads(it, b):
            b0 = s * TBLKS_H + it * gch
            pltpu.async_copy(src_hbm.at[pl.ds(b0, gch)], sidx.at[b], sem_i[b])
            pltpu.async_copy(m_hbm.at[pl.ds(b0 * GB, gch * GB)],
                             mbuf.at[b], sem_i[b])

        def drain_scatter(b):
            for j in range(gch):
                pltpu.make_async_copy(
                    mbuf.at[b].at[pl.ds(j * GB, GB)],
                    accsh.at[pl.ds(0, GB)], sem_s[b]).wait()

        fire_loads(0, 0)

        def body(g, carry):
            for b in (0, 1):
                b2 = 1 - b
                it = 2 * g + b
                pltpu.make_async_copy(
                    src_hbm.at[pl.ds(0, gch)], sidx.at[b], sem_i[b]).wait()
                pltpu.make_async_copy(
                    m_hbm.at[pl.ds(0, gch * GB)], mbuf.at[b], sem_i[b]).wait()

                for j in range(gch):
                    for k in range(GB // 16):
                        v = sidx[b, j, pl.ds(k * 16, 16)]
                        lo = v - base
                        ok = (lo >= 0) & (lo < HALF)
                        lidx[b, j, pl.ds(k * 16, 16)] = jnp.where(ok, lo, HALF)

                if b == 1:
                    drain_scatter(b2)
                else:
                    @pl.when(g > 0)
                    def _drain_prev():
                        drain_scatter(b2)

                for j in range(gch):
                    pltpu.async_copy(
                        mbuf.at[b].at[pl.ds(j * GB, GB)],
                        accsh.at[lidx.at[b].at[j]], sem_s[b], add=True)

                if b == 0:
                    fire_loads(it + 1, b2)
                else:
                    @pl.when(g < s2 - 1)
                    def _fire_next():
                        fire_loads(it + 1, b2)
            return carry

        lax.fori_loop(0, s2, body, 0)

        drain_scatter(1)

        plsc.subcore_barrier()
        pltpu.sync_copy(accsh.at[pl.ds(s * STRIPE, STRIPE)],
                        acc_hbm.at[pl.ds(base + s * STRIPE, STRIPE)])

    return _scatter_k


_scatter_m = _make_scatter(MW, 1)
_scatter_w = _make_scatter(WW, 2)


def kernel(node_feat, coords, edge_attr, edge_index,
           W1, W2, Wemb, Wout, b1, b2, bemb, bout,
           cW1_0, cW1_1, cW2_0, cW2_1, cb1_0, cb1_1,
           eW1_0, eW1_1, eW2_0, eW2_1, eb1_0, eb1_1, eb2_0, eb2_1,
           nW1_0, nW1_1, nW2_0, nW2_1, nb1_0, nb1_1, nb2_0, nb2_1):
    nf_t = jnp.pad(node_feat.T, ((0, 0), (0, NP - N)))
    xyz_t = jnp.pad(coords.T, ((0, 0), (0, NP - N)))
    src_w = jnp.pad(edge_index[0], (0, E_PAD - E),
                    constant_values=PAD_IDX).reshape(NWRK, WBLKS, GB)
    dst_w = jnp.pad(edge_index[1], (0, E_PAD - E),
                    constant_values=PAD_IDX).reshape(NWRK, WBLKS, GB)
    ea_w = jnp.pad(edge_attr.T, ((0, 0), (0, E_PAD - E))
                   ).reshape(2, NWRK, WBLKS, GB)
    halves = []
    for hh in range(2):
        sl = slice(hh * WBLKS_H, (hh + 1) * WBLKS_H)
        halves.append((src_w[:, sl].reshape(HB, GB),
                       dst_w[:, sl].reshape(HB, GB),
                       ea_w[:, :, sl].reshape(2, EH)))
    zeros_m = jnp.zeros((STRIPE, MW), jnp.float32)
    zeros_w = jnp.zeros((STRIPE, WW), jnp.float32)

    def row(b):
        return b.reshape(1, -1)

    t = _front_call(nf_t, xyz_t, W1, row(b1), W2, row(b2), Wemb, row(bemb))

    layers = [
        (eW1_0, eb1_0, eW2_0, eb2_0, cW1_0, cb1_0, cW2_0,
         nW1_0, nb1_0, nW2_0, nb2_0),
        (eW1_1, eb1_1, eW2_1, eb2_1, cW1_1, cb1_1, cW2_1,
         nW1_1, nb1_1, nW2_1, nb2_1),
    ]
    for (ew1, eb1, ew2, eb2, cw1, cb1, cw2, nw1, nb1, nw2, nb2) in layers:
        accs = []
        for (src_h, dst_h, ea_h) in halves:
            gs, gd = _gather_k(t, src_h, dst_h)
            m, w = _edge_call(gs, gd, ea_h,
                              ew1[0:H], ew1[H:2 * H], ew1[2 * H:2 * H + 1],
                              ew1[2 * H + 1:], row(eb1), ew2, row(eb2),
                              cw1, row(cb1), cw2)
            accs.append(_scatter_m(m, src_h, zeros_m))
            accs.append(_scatter_w(w, src_h, zeros_w))
        t = _node_call(t, accs[0], accs[2], accs[1], accs[3],
                       nw1, row(nb1), nw2, row(nb2))

    return _out_call(t, Wout, row(bout))
